# Initial kernel scaffold; baseline (speedup 1.0000x reference)
import jax
import jax.numpy as jnp
from jax.experimental import pallas as pl


def kernel(node_feats, edge_feats, edge_index, gc_pn_W, gc_pn_b, gc_pe1_W, gc_pe1_b, gc_pe2_W, gc_pe2_b, gc_et_W, gc_et_b, gc_gru_Wi, gc_gru_Wh, gc_gru_bi, gc_gru_bh, l1_pe_W, l1_pe_b, l1_pn_W, l1_pn_b, l1_gru_Wi, l1_gru_Wh, l1_gru_bi, l1_gru_bh):
    raise NotImplementedError("write your pallas kernel here")


# speedup vs baseline: 3.0271x; 3.0271x over previous
import functools

import jax
import jax.numpy as jnp
from jax import lax
from jax.experimental import pallas as pl
from jax.experimental.pallas import tpu as pltpu
from jax.experimental.pallas import tpu_sc as plsc

NN = 10000
NE = 320000
DN = 128
DE = 16
DG = 200
DP = 208
NBLK = 1000
EBLK = 512
NCORE = 2
NSUB = 16
NW = NCORE * NSUB
CH = 80


def _leaky(x):
    return jnp.where(x >= 0, x, 0.01 * x)


def _elu(x):
    return jnp.where(x > 0, x, jnp.exp(x) - 1.0)


def _make_gather(n_rows, d):
    per_w = NE // NW
    nloop = per_w // CH
    mesh = plsc.VectorSubcoreMesh(core_axis_name="c", subcore_axis_name="s")

    @functools.partial(
        pl.kernel, mesh=mesh,
        out_type=jax.ShapeDtypeStruct((NE, d), jnp.float32),
        compiler_params=pltpu.CompilerParams(use_tc_tiling_on_sc=False),
        scratch_types=[
            pltpu.VMEM((CH,), jnp.int32),
            pltpu.VMEM((CH, d), jnp.float32),
            pltpu.SemaphoreType.DMA,
        ],
    )
    def gk(tab_hbm, idx_hbm, out_hbm, idx_v, rows_v, sem):
        wid = lax.axis_index("s") * NCORE + lax.axis_index("c")

        def body(t, carry):
            base = wid * per_w + t * CH
            pltpu.sync_copy(idx_hbm.at[pl.ds(base, CH)], idx_v)
            pltpu.async_copy(tab_hbm.at[idx_v], rows_v, sem).wait()
            pltpu.sync_copy(rows_v, out_hbm.at[pl.ds(base, CH)])
            return carry

        lax.fori_loop(0, nloop, body, 0)

    return gk


HALF = NN // NCORE
ACC = HALF + 8


def _make_scatter_add():
    per_s = NE // NSUB
    nloop = per_s // CH
    dump = 200
    ndump = HALF // dump
    mesh = plsc.VectorSubcoreMesh(core_axis_name="c", subcore_axis_name="s")

    @functools.partial(
        pl.kernel, mesh=mesh,
        out_type=jax.ShapeDtypeStruct((NN, DP), jnp.float32),
        compiler_params=pltpu.CompilerParams(use_tc_tiling_on_sc=False),
        scratch_types=[
            pltpu.VMEM((CH,), jnp.int32),
            pltpu.VMEM((CH, DP), jnp.float32),
            pltpu.VMEM_SHARED((ACC, DP), jnp.float32),
        ],
    )
    def sk(rows_hbm, dst_hbm, zero_hbm, out_hbm, idx_v, rows_v, acc_sh):
        cid = lax.axis_index("c")
        sid = lax.axis_index("s")
        lo = cid * HALF

        @pl.when(sid == 0)
        def _():
            pltpu.sync_copy(zero_hbm, acc_sh)

        plsc.subcore_barrier()

        def body(t, carry):
            base = sid * per_s + t * CH
            pltpu.sync_copy(dst_hbm.at[pl.ds(base, CH)], idx_v)
            pltpu.sync_copy(rows_hbm.at[pl.ds(base, CH)], rows_v)
            for k in range(CH // 16):
                v = idx_v[pl.ds(k * 16, 16)] - lo
                ok = (v >= 0) & (v < HALF)
                idx_v[pl.ds(k * 16, 16)] = jnp.where(ok, v, HALF)
            pltpu.sync_copy(rows_v, acc_sh.at[idx_v], add=True)
            return carry

        lax.fori_loop(0, nloop, body, 0)
        plsc.subcore_barrier()

        for t in range((ndump + NSUB - 1) // NSUB):
            j = sid + NSUB * t

            @pl.when(j < ndump)
            def _():
                pltpu.sync_copy(
                    acc_sh.at[pl.ds(j * dump, dump)],
                    out_hbm.at[pl.ds(lo + j * dump, dump)])

    return sk


def _node_pre_body(nf, pnW, pnb, Wn, w1, hv_o, p_o, s1_o):
    hv = _leaky(jnp.dot(nf[...], pnW[...],
                        preferred_element_type=jnp.float32) + pnb[...])
    hv_o[...] = hv
    p_o[...] = jnp.dot(nf[...], Wn[...], preferred_element_type=jnp.float32)
    s1 = jnp.dot(hv, w1[...], preferred_element_type=jnp.float32)
    s1_o[...] = jnp.concatenate(
        [s1, jnp.zeros((s1.shape[0], 7), jnp.float32)], axis=1)


def _edge1_body(ps, ef, s1d, We, b1, w2, b2, etW, etb, out):
    he1 = _leaky(ps[...] + jnp.dot(ef[...], We[...],
                                   preferred_element_type=jnp.float32)
                 + b1[...])
    t = jnp.dot(he1, w2[...], preferred_element_type=jnp.float32)
    w = jnp.exp(_leaky(t + s1d[:, 0:1] + b2[0, 0]))
    m = jnp.dot(he1, etW[...], preferred_element_type=jnp.float32) + etb[...]
    out[...] = jnp.concatenate(
        [w * m, w, jnp.zeros((w.shape[0], 7), jnp.float32)], axis=1)


def _gru_mid_body(part, hv, Wi, Wh, bi, bh, pnW, pnb, u1, u2,
                  h_o, t2_o, q1_o):
    p = part[...]
    num = p[:, :DG]
    den = p[:, DG:DG + 1]
    c = jnp.where(den > 0, num / den, 0.0)
    x = _elu(c)
    gi = jnp.dot(x, Wi[...], preferred_element_type=jnp.float32) + bi[...]
    gh = jnp.dot(hv[...], Wh[...], preferred_element_type=jnp.float32) + bh[...]
    r = jax.nn.sigmoid(gi[:, :DG] + gh[:, :DG])
    z = jax.nn.sigmoid(gi[:, DG:2 * DG] + gh[:, DG:2 * DG])
    n = jnp.tanh(gi[:, 2 * DG:] + r * gh[:, 2 * DG:])
    h = jnp.maximum((1.0 - z) * n + z * hv[...], 0.0)
    h_o[...] = h
    hp = jnp.dot(h, pnW[...], preferred_element_type=jnp.float32) + pnb[...]
    q2 = jnp.dot(h, u2[...], preferred_element_type=jnp.float32)
    q1 = jnp.dot(h, u1[...], preferred_element_type=jnp.float32)
    t2_o[...] = jnp.concatenate(
        [hp, q2, jnp.zeros((hp.shape[0], 7), jnp.float32)], axis=1)
    q1_o[...] = jnp.concatenate(
        [q1, jnp.zeros((q1.shape[0], 7), jnp.float32)], axis=1)


def _edge2_body(hs, q1d, b2, out):
    h = hs[...]
    w = jnp.exp(_leaky(h[:, DG:DG + 1] + q1d[:, 0:1] + b2[0, 0]))
    out[...] = jnp.concatenate(
        [w * h[:, :DG], w, jnp.zeros((w.shape[0], 7), jnp.float32)], axis=1)


def _gru_fin_body(part, hv, Wi, Wh, bi, bh, h_o):
    p = part[...]
    num = p[:, :DG]
    den = p[:, DG:DG + 1]
    c = jnp.where(den > 0, num / den, 0.0)
    x = _elu(c)
    gi = jnp.dot(x, Wi[...], preferred_element_type=jnp.float32) + bi[...]
    gh = jnp.dot(hv[...], Wh[...], preferred_element_type=jnp.float32) + bh[...]
    r = jax.nn.sigmoid(gi[:, :DG] + gh[:, :DG])
    z = jax.nn.sigmoid(gi[:, DG:2 * DG] + gh[:, DG:2 * DG])
    n = jnp.tanh(gi[:, 2 * DG:] + r * gh[:, 2 * DG:])
    h_o[...] = jnp.maximum((1.0 - z) * n + z * hv[...], 0.0)


def _full(shape):
    return pl.BlockSpec(shape, lambda i: tuple(0 for _ in shape))


def _rows(shape):
    return pl.BlockSpec(shape, lambda i: (i,) + tuple(0 for _ in shape[1:]))


def kernel(node_feats, edge_feats, edge_index, gc_pn_W, gc_pn_b, gc_pe1_W,
           gc_pe1_b, gc_pe2_W, gc_pe2_b, gc_et_W, gc_et_b, gc_gru_Wi,
           gc_gru_Wh, gc_gru_bi, gc_gru_bh, l1_pe_W, l1_pe_b, l1_pn_W,
           l1_pn_b, l1_gru_Wi, l1_gru_Wh, l1_gru_bi, l1_gru_bh):
    f32 = jnp.float32
    src = edge_index[0]
    dst = edge_index[1]
    Wn = gc_pe1_W[:DN]
    We = gc_pe1_W[DN:]
    w1 = gc_pe2_W[:DG]
    w2 = gc_pe2_W[DG:]
    u1 = l1_pe_W[:DG]
    u2 = l1_pe_W[DG:]
    b1 = gc_pe1_b.reshape(1, DG)
    b2 = gc_pe2_b.reshape(1, 1)
    pnb = gc_pn_b.reshape(1, DG)
    etb = gc_et_b.reshape(1, DG)
    bi1 = gc_gru_bi.reshape(1, 3 * DG)
    bh1 = gc_gru_bh.reshape(1, 3 * DG)
    l1b = l1_pe_b.reshape(1, 1)
    l1pnb = l1_pn_b.reshape(1, DG)
    bi2 = l1_gru_bi.reshape(1, 3 * DG)
    bh2 = l1_gru_bh.reshape(1, 3 * DG)
    zero_tab = jnp.zeros((ACC, DP), f32)

    ng = NN // NBLK
    eg = NE // EBLK

    hv_new, ptab, s1tab = pl.pallas_call(
        _node_pre_body,
        grid=(ng,),
        in_specs=[_rows((NBLK, DN)), _full((DN, DG)), _full((1, DG)),
                  _full((DN, DG)), _full((DG, 1))],
        out_specs=[_rows((NBLK, DG)), _rows((NBLK, DG)), _rows((NBLK, 8))],
        out_shape=[jax.ShapeDtypeStruct((NN, DG), f32),
                   jax.ShapeDtypeStruct((NN, DG), f32),
                   jax.ShapeDtypeStruct((NN, 8), f32)],
    )(node_feats, gc_pn_W, pnb, Wn, w1)

    psrc = _make_gather(NN, DG)(ptab, src)
    s1d = _make_gather(NN, 8)(s1tab, dst)

    rows1 = pl.pallas_call(
        _edge1_body,
        grid=(eg,),
        in_specs=[_rows((EBLK, DG)), _rows((EBLK, DE)), _rows((EBLK, 8)),
                  _full((DE, DG)), _full((1, DG)), _full((DG, 1)),
                  _full((1, 1)), _full((DG, DG)), _full((1, DG))],
        out_specs=_rows((EBLK, DP)),
        out_shape=jax.ShapeDtypeStruct((NE, DP), f32),
    )(psrc, edge_feats, s1d, We, b1, w2, b2, gc_et_W, etb)

    part1 = _make_scatter_add()(rows1, dst, zero_tab)

    h, t2tab, q1tab = pl.pallas_call(
        _gru_mid_body,
        grid=(ng,),
        in_specs=[_rows((NBLK, DP)),
                  _rows((NBLK, DG)),
                  _full((DG, 3 * DG)), _full((DG, 3 * DG)),
                  _full((1, 3 * DG)), _full((1, 3 * DG)),
                  _full((DG, DG)), _full((1, DG)),
                  _full((DG, 1)), _full((DG, 1))],
        out_specs=[_rows((NBLK, DG)), _rows((NBLK, DP)), _rows((NBLK, 8))],
        out_shape=[jax.ShapeDtypeStruct((NN, DG), f32),
                   jax.ShapeDtypeStruct((NN, DP), f32),
                   jax.ShapeDtypeStruct((NN, 8), f32)],
    )(part1, hv_new, gc_gru_Wi, gc_gru_Wh, bi1, bh1, l1_pn_W, l1pnb, u1, u2)

    hsrc = _make_gather(NN, DP)(t2tab, src)
    q1d = _make_gather(NN, 8)(q1tab, dst)

    rows2 = pl.pallas_call(
        _edge2_body,
        grid=(eg,),
        in_specs=[_rows((EBLK, DP)), _rows((EBLK, 8)), _full((1, 1))],
        out_specs=_rows((EBLK, DP)),
        out_shape=jax.ShapeDtypeStruct((NE, DP), f32),
    )(hsrc, q1d, l1b)

    part2 = _make_scatter_add()(rows2, dst, zero_tab)

    h2 = pl.pallas_call(
        _gru_fin_body,
        grid=(ng,),
        in_specs=[_rows((NBLK, DP)),
                  _rows((NBLK, DG)),
                  _full((DG, 3 * DG)), _full((DG, 3 * DG)),
                  _full((1, 3 * DG)), _full((1, 3 * DG))],
        out_specs=_rows((NBLK, DG)),
        out_shape=jax.ShapeDtypeStruct((NN, DG), f32),
    )(part2, h, l1_gru_Wi, l1_gru_Wh, bi2, bh2)

    return h2.reshape(NN // 100, 100, DG)


# speedup vs baseline: 3.2890x; 1.0865x over previous
import functools

import jax
import jax.numpy as jnp
from jax import lax
from jax.experimental import pallas as pl
from jax.experimental.pallas import tpu as pltpu
from jax.experimental.pallas import tpu_sc as plsc

NN = 10000
NE = 320000
DN = 128
DE = 16
DG = 200
DP = 208
NBLK = 1000
EBLK = 512
NCORE = 2
NSUB = 16
NW = NCORE * NSUB
CH = 80


def _leaky(x):
    return jnp.where(x >= 0, x, 0.01 * x)


def _elu(x):
    return jnp.where(x > 0, x, jnp.exp(x) - 1.0)


def _make_gather(n_rows, d):
    per_w = NE // NW
    nloop = per_w // CH
    mesh = plsc.VectorSubcoreMesh(core_axis_name="c", subcore_axis_name="s")

    kf = 5
    grp = kf * CH
    ngrp = per_w // grp

    @functools.partial(
        pl.kernel, mesh=mesh,
        out_type=jax.ShapeDtypeStruct((NE, d), jnp.float32),
        compiler_params=pltpu.CompilerParams(use_tc_tiling_on_sc=False),
        scratch_types=[
            pltpu.VMEM((per_w,), jnp.int32),
            pltpu.VMEM((grp, d), jnp.float32),
            pltpu.SemaphoreType.DMA,
        ],
    )
    def gk(tab_hbm, idx_hbm, out_hbm, idx_v, rows_v, sem):
        wid = lax.axis_index("s") * NCORE + lax.axis_index("c")
        pltpu.sync_copy(idx_hbm.at[pl.ds(wid * per_w, per_w)], idx_v)

        def body(g, carry):
            waits = []
            for i in range(kf):
                waits.append(pltpu.async_copy(
                    tab_hbm.at[idx_v.at[pl.ds(g * grp + i * CH, CH)]],
                    rows_v.at[pl.ds(i * CH, CH)], sem))
            for w in waits:
                w.wait()
            pltpu.sync_copy(
                rows_v, out_hbm.at[pl.ds(wid * per_w + g * grp, grp)])
            return carry

        lax.fori_loop(0, ngrp, body, 0)

    return gk


HALF = NN // NCORE
ACC = HALF + 8


def _make_scatter_add():
    per_s = NE // NSUB
    nloop = per_s // CH
    dump = 200
    ndump = HALF // dump
    mesh = plsc.VectorSubcoreMesh(core_axis_name="c", subcore_axis_name="s")

    @functools.partial(
        pl.kernel, mesh=mesh,
        out_type=jax.ShapeDtypeStruct((NN, DP), jnp.float32),
        compiler_params=pltpu.CompilerParams(use_tc_tiling_on_sc=False),
        scratch_types=[
            pltpu.VMEM((CH,), jnp.int32),
            pltpu.VMEM((CH, DP), jnp.float32),
            pltpu.VMEM_SHARED((ACC, DP), jnp.float32),
        ],
    )
    def sk(rows_hbm, dst_hbm, zero_hbm, out_hbm, idx_v, rows_v, acc_sh):
        cid = lax.axis_index("c")
        sid = lax.axis_index("s")
        lo = cid * HALF

        @pl.when(sid == 0)
        def _():
            pltpu.sync_copy(zero_hbm, acc_sh)

        plsc.subcore_barrier()

        def body(t, carry):
            base = sid * per_s + t * CH
            pltpu.sync_copy(dst_hbm.at[pl.ds(base, CH)], idx_v)
            pltpu.sync_copy(rows_hbm.at[pl.ds(base, CH)], rows_v)
            for k in range(CH // 16):
                v = idx_v[pl.ds(k * 16, 16)] - lo
                ok = (v >= 0) & (v < HALF)
                idx_v[pl.ds(k * 16, 16)] = jnp.where(ok, v, HALF)
            pltpu.sync_copy(rows_v, acc_sh.at[idx_v], add=True)
            return carry

        lax.fori_loop(0, nloop, body, 0)
        plsc.subcore_barrier()

        for t in range((ndump + NSUB - 1) // NSUB):
            j = sid + NSUB * t

            @pl.when(j < ndump)
            def _():
                pltpu.sync_copy(
                    acc_sh.at[pl.ds(j * dump, dump)],
                    out_hbm.at[pl.ds(lo + j * dump, dump)])

    return sk


def _node_pre_body(nf, pnW, pnb, Wn, w1, hv_o, p_o, s1_o):
    hv = _leaky(jnp.dot(nf[...], pnW[...],
                        preferred_element_type=jnp.float32) + pnb[...])
    hv_o[...] = hv
    p_o[...] = jnp.dot(nf[...], Wn[...], preferred_element_type=jnp.float32)
    s1 = jnp.dot(hv, w1[...], preferred_element_type=jnp.float32)
    s1_o[...] = jnp.concatenate(
        [s1, jnp.zeros((s1.shape[0], 7), jnp.float32)], axis=1)


def _edge1_body(ps, ef, s1d, We, b1, w2, b2, etW, etb, out):
    he1 = _leaky(ps[...] + jnp.dot(ef[...], We[...],
                                   preferred_element_type=jnp.float32)
                 + b1[...])
    t = jnp.dot(he1, w2[...], preferred_element_type=jnp.float32)
    w = jnp.exp(_leaky(t + s1d[:, 0:1] + b2[0, 0]))
    m = jnp.dot(he1, etW[...], preferred_element_type=jnp.float32) + etb[...]
    out[...] = jnp.concatenate(
        [w * m, w, jnp.zeros((w.shape[0], 7), jnp.float32)], axis=1)


def _gru_mid_body(part, hv, Wi, Wh, bi, bh, pnW, pnb, u1, u2,
                  h_o, t2_o, q1_o):
    p = part[...]
    num = p[:, :DG]
    den = p[:, DG:DG + 1]
    c = jnp.where(den > 0, num / den, 0.0)
    x = _elu(c)
    gi = jnp.dot(x, Wi[...], preferred_element_type=jnp.float32) + bi[...]
    gh = jnp.dot(hv[...], Wh[...], preferred_element_type=jnp.float32) + bh[...]
    r = jax.nn.sigmoid(gi[:, :DG] + gh[:, :DG])
    z = jax.nn.sigmoid(gi[:, DG:2 * DG] + gh[:, DG:2 * DG])
    n = jnp.tanh(gi[:, 2 * DG:] + r * gh[:, 2 * DG:])
    h = jnp.maximum((1.0 - z) * n + z * hv[...], 0.0)
    h_o[...] = h
    hp = jnp.dot(h, pnW[...], preferred_element_type=jnp.float32) + pnb[...]
    q2 = jnp.dot(h, u2[...], preferred_element_type=jnp.float32)
    q1 = jnp.dot(h, u1[...], preferred_element_type=jnp.float32)
    t2_o[...] = jnp.concatenate(
        [hp, q2, jnp.zeros((hp.shape[0], 7), jnp.float32)], axis=1)
    q1_o[...] = jnp.concatenate(
        [q1, jnp.zeros((q1.shape[0], 7), jnp.float32)], axis=1)


def _edge2_body(hs, q1d, b2, out):
    h = hs[...]
    w = jnp.exp(_leaky(h[:, DG:DG + 1] + q1d[:, 0:1] + b2[0, 0]))
    out[...] = jnp.concatenate(
        [w * h[:, :DG], w, jnp.zeros((w.shape[0], 7), jnp.float32)], axis=1)


def _gru_fin_body(part, hv, Wi, Wh, bi, bh, h_o):
    p = part[...]
    num = p[:, :DG]
    den = p[:, DG:DG + 1]
    c = jnp.where(den > 0, num / den, 0.0)
    x = _elu(c)
    gi = jnp.dot(x, Wi[...], preferred_element_type=jnp.float32) + bi[...]
    gh = jnp.dot(hv[...], Wh[...], preferred_element_type=jnp.float32) + bh[...]
    r = jax.nn.sigmoid(gi[:, :DG] + gh[:, :DG])
    z = jax.nn.sigmoid(gi[:, DG:2 * DG] + gh[:, DG:2 * DG])
    n = jnp.tanh(gi[:, 2 * DG:] + r * gh[:, 2 * DG:])
    h_o[...] = jnp.maximum((1.0 - z) * n + z * hv[...], 0.0)


def _full(shape):
    return pl.BlockSpec(shape, lambda i: tuple(0 for _ in shape))


def _rows(shape):
    return pl.BlockSpec(shape, lambda i: (i,) + tuple(0 for _ in shape[1:]))


def kernel(node_feats, edge_feats, edge_index, gc_pn_W, gc_pn_b, gc_pe1_W,
           gc_pe1_b, gc_pe2_W, gc_pe2_b, gc_et_W, gc_et_b, gc_gru_Wi,
           gc_gru_Wh, gc_gru_bi, gc_gru_bh, l1_pe_W, l1_pe_b, l1_pn_W,
           l1_pn_b, l1_gru_Wi, l1_gru_Wh, l1_gru_bi, l1_gru_bh):
    f32 = jnp.float32
    src = edge_index[0]
    dst = edge_index[1]
    Wn = gc_pe1_W[:DN]
    We = gc_pe1_W[DN:]
    w1 = gc_pe2_W[:DG]
    w2 = gc_pe2_W[DG:]
    u1 = l1_pe_W[:DG]
    u2 = l1_pe_W[DG:]
    b1 = gc_pe1_b.reshape(1, DG)
    b2 = gc_pe2_b.reshape(1, 1)
    pnb = gc_pn_b.reshape(1, DG)
    etb = gc_et_b.reshape(1, DG)
    bi1 = gc_gru_bi.reshape(1, 3 * DG)
    bh1 = gc_gru_bh.reshape(1, 3 * DG)
    l1b = l1_pe_b.reshape(1, 1)
    l1pnb = l1_pn_b.reshape(1, DG)
    bi2 = l1_gru_bi.reshape(1, 3 * DG)
    bh2 = l1_gru_bh.reshape(1, 3 * DG)
    zero_tab = jnp.zeros((ACC, DP), f32)

    ng = NN // NBLK
    eg = NE // EBLK

    hv_new, ptab, s1tab = pl.pallas_call(
        _node_pre_body,
        grid=(ng,),
        in_specs=[_rows((NBLK, DN)), _full((DN, DG)), _full((1, DG)),
                  _full((DN, DG)), _full((DG, 1))],
        out_specs=[_rows((NBLK, DG)), _rows((NBLK, DG)), _rows((NBLK, 8))],
        out_shape=[jax.ShapeDtypeStruct((NN, DG), f32),
                   jax.ShapeDtypeStruct((NN, DG), f32),
                   jax.ShapeDtypeStruct((NN, 8), f32)],
    )(node_feats, gc_pn_W, pnb, Wn, w1)

    psrc = _make_gather(NN, DG)(ptab, src)
    s1d = _make_gather(NN, 8)(s1tab, dst)

    rows1 = pl.pallas_call(
        _edge1_body,
        grid=(eg,),
        in_specs=[_rows((EBLK, DG)), _rows((EBLK, DE)), _rows((EBLK, 8)),
                  _full((DE, DG)), _full((1, DG)), _full((DG, 1)),
                  _full((1, 1)), _full((DG, DG)), _full((1, DG))],
        out_specs=_rows((EBLK, DP)),
        out_shape=jax.ShapeDtypeStruct((NE, DP), f32),
    )(psrc, edge_feats, s1d, We, b1, w2, b2, gc_et_W, etb)

    part1 = _make_scatter_add()(rows1, dst, zero_tab)

    h, t2tab, q1tab = pl.pallas_call(
        _gru_mid_body,
        grid=(ng,),
        in_specs=[_rows((NBLK, DP)),
                  _rows((NBLK, DG)),
                  _full((DG, 3 * DG)), _full((DG, 3 * DG)),
                  _full((1, 3 * DG)), _full((1, 3 * DG)),
                  _full((DG, DG)), _full((1, DG)),
                  _full((DG, 1)), _full((DG, 1))],
        out_specs=[_rows((NBLK, DG)), _rows((NBLK, DP)), _rows((NBLK, 8))],
        out_shape=[jax.ShapeDtypeStruct((NN, DG), f32),
                   jax.ShapeDtypeStruct((NN, DP), f32),
                   jax.ShapeDtypeStruct((NN, 8), f32)],
    )(part1, hv_new, gc_gru_Wi, gc_gru_Wh, bi1, bh1, l1_pn_W, l1pnb, u1, u2)

    hsrc = _make_gather(NN, DP)(t2tab, src)
    q1d = _make_gather(NN, 8)(q1tab, dst)

    rows2 = pl.pallas_call(
        _edge2_body,
        grid=(eg,),
        in_specs=[_rows((EBLK, DP)), _rows((EBLK, 8)), _full((1, 1))],
        out_specs=_rows((EBLK, DP)),
        out_shape=jax.ShapeDtypeStruct((NE, DP), f32),
    )(hsrc, q1d, l1b)

    part2 = _make_scatter_add()(rows2, dst, zero_tab)

    h2 = pl.pallas_call(
        _gru_fin_body,
        grid=(ng,),
        in_specs=[_rows((NBLK, DP)),
                  _rows((NBLK, DG)),
                  _full((DG, 3 * DG)), _full((DG, 3 * DG)),
                  _full((1, 3 * DG)), _full((1, 3 * DG))],
        out_specs=_rows((NBLK, DG)),
        out_shape=jax.ShapeDtypeStruct((NN, DG), f32),
    )(part2, h, l1_gru_Wi, l1_gru_Wh, bi2, bh2)

    return h2.reshape(NN // 100, 100, DG)


# speedup vs baseline: 3.5317x; 1.0738x over previous
import functools

import jax
import jax.numpy as jnp
from jax import lax
from jax.experimental import pallas as pl
from jax.experimental.pallas import tpu as pltpu
from jax.experimental.pallas import tpu_sc as plsc

NN = 10000
NE = 320000
DN = 128
DE = 16
DG = 200
DP = 208
NBLK = 1000
EBLK = 512
NCORE = 2
NSUB = 16
NW = NCORE * NSUB
CH = 80


def _leaky(x):
    return jnp.where(x >= 0, x, 0.01 * x)


def _elu(x):
    return jnp.where(x > 0, x, jnp.exp(x) - 1.0)


def _make_gather(n_rows, d):
    per_w = NE // NW
    nloop = per_w // CH
    mesh = plsc.VectorSubcoreMesh(core_axis_name="c", subcore_axis_name="s")

    kf = 5
    grp = kf * CH
    ngrp = per_w // grp

    @functools.partial(
        pl.kernel, mesh=mesh,
        out_type=jax.ShapeDtypeStruct((NE, d), jnp.float32),
        compiler_params=pltpu.CompilerParams(use_tc_tiling_on_sc=False),
        scratch_types=[
            pltpu.VMEM((per_w,), jnp.int32),
            pltpu.VMEM((grp, d), jnp.float32),
            pltpu.SemaphoreType.DMA,
        ],
    )
    def gk(tab_hbm, idx_hbm, out_hbm, idx_v, rows_v, sem):
        wid = lax.axis_index("s") * NCORE + lax.axis_index("c")
        pltpu.sync_copy(idx_hbm.at[pl.ds(wid * per_w, per_w)], idx_v)

        def body(g, carry):
            waits = []
            for i in range(kf):
                waits.append(pltpu.async_copy(
                    tab_hbm.at[idx_v.at[pl.ds(g * grp + i * CH, CH)]],
                    rows_v.at[pl.ds(i * CH, CH)], sem))
            for w in waits:
                w.wait()
            pltpu.sync_copy(
                rows_v, out_hbm.at[pl.ds(wid * per_w + g * grp, grp)])
            return carry

        lax.fori_loop(0, ngrp, body, 0)

    return gk


HALF = NN // NCORE
ACC = HALF + 8


def _make_scatter_add():
    per_s = NE // NSUB
    nloop = per_s // CH
    dump = 200
    ndump = HALF // dump
    mesh = plsc.VectorSubcoreMesh(core_axis_name="c", subcore_axis_name="s")

    @functools.partial(
        pl.kernel, mesh=mesh,
        out_type=jax.ShapeDtypeStruct((NN, DP), jnp.float32),
        compiler_params=pltpu.CompilerParams(use_tc_tiling_on_sc=False),
        scratch_types=[
            pltpu.VMEM((2, CH), jnp.int32),
            pltpu.VMEM((2, CH, DP), jnp.float32),
            pltpu.VMEM_SHARED((ACC, DP), jnp.float32),
            pltpu.SemaphoreType.DMA,
            pltpu.SemaphoreType.DMA,
        ],
    )
    def sk(rows_hbm, dst_hbm, zero_hbm, out_hbm, idx_v, rows_v, acc_sh,
           sem0, sem1):
        cid = lax.axis_index("c")
        sid = lax.axis_index("s")
        lo = cid * HALF
        sems = (sem0, sem1)

        @pl.when(sid == 0)
        def _():
            pltpu.sync_copy(zero_hbm, acc_sh)

        plsc.subcore_barrier()

        def fire(t, b):
            base = sid * per_s + t * CH
            wi = pltpu.async_copy(
                dst_hbm.at[pl.ds(base, CH)], idx_v.at[b], sems[b])
            wr = pltpu.async_copy(
                rows_hbm.at[pl.ds(base, CH)], rows_v.at[b], sems[b])
            return wi, wr

        def drain_and_scatter(b, waits):
            for w in waits:
                w.wait()
            for k in range(CH // 16):
                v = idx_v[b, pl.ds(k * 16, 16)] - lo
                ok = (v >= 0) & (v < HALF)
                idx_v[b, pl.ds(k * 16, 16)] = jnp.where(ok, v, HALF)
            pltpu.sync_copy(rows_v.at[b], acc_sh.at[idx_v.at[b]], add=True)

        def pair(g, carry):
            t = g * 2
            wa = fire(t, 0)
            wb = fire(t + 1, 1)
            drain_and_scatter(0, wa)
            drain_and_scatter(1, wb)
            return carry

        lax.fori_loop(0, nloop // 2, pair, 0)
        plsc.subcore_barrier()

        for t in range((ndump + NSUB - 1) // NSUB):
            j = sid + NSUB * t

            @pl.when(j < ndump)
            def _():
                pltpu.sync_copy(
                    acc_sh.at[pl.ds(j * dump, dump)],
                    out_hbm.at[pl.ds(lo + j * dump, dump)])

    return sk


def _node_pre_body(nf, pnW, pnb, Wn, w1, hv_o, p_o, s1_o):
    hv = _leaky(jnp.dot(nf[...], pnW[...],
                        preferred_element_type=jnp.float32) + pnb[...])
    hv_o[...] = hv
    p_o[...] = jnp.dot(nf[...], Wn[...], preferred_element_type=jnp.float32)
    s1 = jnp.dot(hv, w1[...], preferred_element_type=jnp.float32)
    s1_o[...] = jnp.concatenate(
        [s1, jnp.zeros((s1.shape[0], 7), jnp.float32)], axis=1)


def _edge1_body(ps, ef, s1d, We, b1, w2, b2, etW, etb, out):
    he1 = _leaky(ps[...] + jnp.dot(ef[...], We[...],
                                   preferred_element_type=jnp.float32)
                 + b1[...])
    t = jnp.dot(he1, w2[...], preferred_element_type=jnp.float32)
    w = jnp.exp(_leaky(t + s1d[:, 0:1] + b2[0, 0]))
    m = jnp.dot(he1, etW[...], preferred_element_type=jnp.float32) + etb[...]
    out[...] = jnp.concatenate(
        [w * m, w, jnp.zeros((w.shape[0], 7), jnp.float32)], axis=1)


def _gru_mid_body(part, hv, Wi, Wh, bi, bh, pnW, pnb, u1, u2,
                  h_o, t2_o, q1_o):
    p = part[...]
    num = p[:, :DG]
    den = p[:, DG:DG + 1]
    c = jnp.where(den > 0, num / den, 0.0)
    x = _elu(c)
    gi = jnp.dot(x, Wi[...], preferred_element_type=jnp.float32) + bi[...]
    gh = jnp.dot(hv[...], Wh[...], preferred_element_type=jnp.float32) + bh[...]
    r = jax.nn.sigmoid(gi[:, :DG] + gh[:, :DG])
    z = jax.nn.sigmoid(gi[:, DG:2 * DG] + gh[:, DG:2 * DG])
    n = jnp.tanh(gi[:, 2 * DG:] + r * gh[:, 2 * DG:])
    h = jnp.maximum((1.0 - z) * n + z * hv[...], 0.0)
    h_o[...] = h
    hp = jnp.dot(h, pnW[...], preferred_element_type=jnp.float32) + pnb[...]
    q2 = jnp.dot(h, u2[...], preferred_element_type=jnp.float32)
    q1 = jnp.dot(h, u1[...], preferred_element_type=jnp.float32)
    t2_o[...] = jnp.concatenate(
        [hp, q2, jnp.zeros((hp.shape[0], 7), jnp.float32)], axis=1)
    q1_o[...] = jnp.concatenate(
        [q1, jnp.zeros((q1.shape[0], 7), jnp.float32)], axis=1)


def _edge2_body(hs, q1d, b2, out):
    h = hs[...]
    w = jnp.exp(_leaky(h[:, DG:DG + 1] + q1d[:, 0:1] + b2[0, 0]))
    out[...] = jnp.concatenate(
        [w * h[:, :DG], w, jnp.zeros((w.shape[0], 7), jnp.float32)], axis=1)


def _gru_fin_body(part, hv, Wi, Wh, bi, bh, h_o):
    p = part[...]
    num = p[:, :DG]
    den = p[:, DG:DG + 1]
    c = jnp.where(den > 0, num / den, 0.0)
    x = _elu(c)
    gi = jnp.dot(x, Wi[...], preferred_element_type=jnp.float32) + bi[...]
    gh = jnp.dot(hv[...], Wh[...], preferred_element_type=jnp.float32) + bh[...]
    r = jax.nn.sigmoid(gi[:, :DG] + gh[:, :DG])
    z = jax.nn.sigmoid(gi[:, DG:2 * DG] + gh[:, DG:2 * DG])
    n = jnp.tanh(gi[:, 2 * DG:] + r * gh[:, 2 * DG:])
    h_o[...] = jnp.maximum((1.0 - z) * n + z * hv[...], 0.0)


def _full(shape):
    return pl.BlockSpec(shape, lambda i: tuple(0 for _ in shape))


def _rows(shape):
    return pl.BlockSpec(shape, lambda i: (i,) + tuple(0 for _ in shape[1:]))


def kernel(node_feats, edge_feats, edge_index, gc_pn_W, gc_pn_b, gc_pe1_W,
           gc_pe1_b, gc_pe2_W, gc_pe2_b, gc_et_W, gc_et_b, gc_gru_Wi,
           gc_gru_Wh, gc_gru_bi, gc_gru_bh, l1_pe_W, l1_pe_b, l1_pn_W,
           l1_pn_b, l1_gru_Wi, l1_gru_Wh, l1_gru_bi, l1_gru_bh):
    f32 = jnp.float32
    src = edge_index[0]
    dst = edge_index[1]
    Wn = gc_pe1_W[:DN]
    We = gc_pe1_W[DN:]
    w1 = gc_pe2_W[:DG]
    w2 = gc_pe2_W[DG:]
    u1 = l1_pe_W[:DG]
    u2 = l1_pe_W[DG:]
    b1 = gc_pe1_b.reshape(1, DG)
    b2 = gc_pe2_b.reshape(1, 1)
    pnb = gc_pn_b.reshape(1, DG)
    etb = gc_et_b.reshape(1, DG)
    bi1 = gc_gru_bi.reshape(1, 3 * DG)
    bh1 = gc_gru_bh.reshape(1, 3 * DG)
    l1b = l1_pe_b.reshape(1, 1)
    l1pnb = l1_pn_b.reshape(1, DG)
    bi2 = l1_gru_bi.reshape(1, 3 * DG)
    bh2 = l1_gru_bh.reshape(1, 3 * DG)
    zero_tab = jnp.zeros((ACC, DP), f32)

    ng = NN // NBLK
    eg = NE // EBLK

    hv_new, ptab, s1tab = pl.pallas_call(
        _node_pre_body,
        grid=(ng,),
        in_specs=[_rows((NBLK, DN)), _full((DN, DG)), _full((1, DG)),
                  _full((DN, DG)), _full((DG, 1))],
        out_specs=[_rows((NBLK, DG)), _rows((NBLK, DG)), _rows((NBLK, 8))],
        out_shape=[jax.ShapeDtypeStruct((NN, DG), f32),
                   jax.ShapeDtypeStruct((NN, DG), f32),
                   jax.ShapeDtypeStruct((NN, 8), f32)],
    )(node_feats, gc_pn_W, pnb, Wn, w1)

    psrc = _make_gather(NN, DG)(ptab, src)
    s1d = _make_gather(NN, 8)(s1tab, dst)

    rows1 = pl.pallas_call(
        _edge1_body,
        grid=(eg,),
        in_specs=[_rows((EBLK, DG)), _rows((EBLK, DE)), _rows((EBLK, 8)),
                  _full((DE, DG)), _full((1, DG)), _full((DG, 1)),
                  _full((1, 1)), _full((DG, DG)), _full((1, DG))],
        out_specs=_rows((EBLK, DP)),
        out_shape=jax.ShapeDtypeStruct((NE, DP), f32),
    )(psrc, edge_feats, s1d, We, b1, w2, b2, gc_et_W, etb)

    part1 = _make_scatter_add()(rows1, dst, zero_tab)

    h, t2tab, q1tab = pl.pallas_call(
        _gru_mid_body,
        grid=(ng,),
        in_specs=[_rows((NBLK, DP)),
                  _rows((NBLK, DG)),
                  _full((DG, 3 * DG)), _full((DG, 3 * DG)),
                  _full((1, 3 * DG)), _full((1, 3 * DG)),
                  _full((DG, DG)), _full((1, DG)),
                  _full((DG, 1)), _full((DG, 1))],
        out_specs=[_rows((NBLK, DG)), _rows((NBLK, DP)), _rows((NBLK, 8))],
        out_shape=[jax.ShapeDtypeStruct((NN, DG), f32),
                   jax.ShapeDtypeStruct((NN, DP), f32),
                   jax.ShapeDtypeStruct((NN, 8), f32)],
    )(part1, hv_new, gc_gru_Wi, gc_gru_Wh, bi1, bh1, l1_pn_W, l1pnb, u1, u2)

    hsrc = _make_gather(NN, DP)(t2tab, src)
    q1d = _make_gather(NN, 8)(q1tab, dst)

    rows2 = pl.pallas_call(
        _edge2_body,
        grid=(eg,),
        in_specs=[_rows((EBLK, DP)), _rows((EBLK, 8)), _full((1, 1))],
        out_specs=_rows((EBLK, DP)),
        out_shape=jax.ShapeDtypeStruct((NE, DP), f32),
    )(hsrc, q1d, l1b)

    part2 = _make_scatter_add()(rows2, dst, zero_tab)

    h2 = pl.pallas_call(
        _gru_fin_body,
        grid=(ng,),
        in_specs=[_rows((NBLK, DP)),
                  _rows((NBLK, DG)),
                  _full((DG, 3 * DG)), _full((DG, 3 * DG)),
                  _full((1, 3 * DG)), _full((1, 3 * DG))],
        out_specs=_rows((NBLK, DG)),
        out_shape=jax.ShapeDtypeStruct((NN, DG), f32),
    )(part2, h, l1_gru_Wi, l1_gru_Wh, bi2, bh2)

    return h2.reshape(NN // 100, 100, DG)
